# baseline (speedup 1.0000x reference)
import jax
import jax.numpy as jnp
from jax import lax
from jax.experimental import pallas as pl
from jax.experimental.pallas import tpu as pltpu
from jax.experimental.pallas import tpu_sc as plsc

_B = 16
_H = 512
_W = 512
_N = 2000
_NWORK = 32
_ROWS_PER_W = (_B * _H) // _NWORK
_CH = 128
_NCHUNK = _ROWS_PER_W // _CH
_NVEC = _N // 16


def _station_loss_body(pred_hbm, tgt_hbm, rows_hbm, cols_hbm, out_hbm,
                       rows_v, cols_v, tgtb_v, slab_v, acc_v,
                       sem, slab_sem):
    cid = lax.axis_index("c")
    sid = lax.axis_index("s")
    wid = sid * 2 + cid
    b = wid // 2
    row0 = wid * _ROWS_PER_W
    h0 = (wid % 2) * _ROWS_PER_W

    def start_chunk(k):
        return [
            pltpu.async_copy(
                pred_hbm.at[pl.ds(row0 + k * _CH + g * 8, 8),
                            pl.ds(ct * 128, 128)],
                slab_v.at[g * 4 + ct], slab_sem)
            for g in range(_CH // 8) for ct in range(_W // 128)
        ]

    cps = start_chunk(0)
    cp_r = pltpu.async_copy(rows_hbm, rows_v, sem)
    cp_c = pltpu.async_copy(cols_hbm, cols_v, sem)
    cp_t = pltpu.async_copy(tgt_hbm.at[pl.ds(b * _N, _N)], tgtb_v, sem)
    cp_r.wait()
    cp_c.wait()
    cp_t.wait()

    def pre_body(i, carry):
        base = i * 16
        r = rows_v[pl.ds(base, 16)]
        c = cols_v[pl.ds(base, 16)]
        rows_v[pl.ds(base, 16)] = ((r - (h0 - _ROWS_PER_W)) << 9) | c
        return carry

    lax.fori_loop(0, _NVEC, pre_body, 0, unroll=4)

    acc = jnp.zeros((16,), jnp.float32)
    for k in range(_NCHUNK):
        if k > 0:
            cps = start_chunk(k)
        for cp in cps:
            cp.wait()

        def sweep_body(i, acc, k=k):
            base = i * 16
            pk = rows_v[pl.ds(base, 16)]
            m = (pk >> 16) == (_ROWS_PER_W >> 7) + k
            tile = ((pk >> 12) & 15) * 4 + ((pk >> 7) & 3)
            x = plsc.load_gather(
                slab_v, [tile, (pk >> 9) & 7, pk & 127], mask=m)
            d = jnp.abs(x - tgtb_v[pl.ds(base, 16)])
            return acc + jnp.where(m, d, 0.0)

        acc = lax.fori_loop(0, _NVEC, sweep_body, acc, unroll=8)

    acc_v[...] = acc
    pltpu.sync_copy(acc_v, out_hbm.at[wid])


def kernel(pred_images, target_runoff_values, station_rows, station_cols):
    pred2 = pred_images.reshape(_B * _H, _W)
    tgt_t = target_runoff_values[:, :_B].T.reshape(-1)

    mesh = plsc.VectorSubcoreMesh(core_axis_name="c", subcore_axis_name="s")
    partials = pl.kernel(
        _station_loss_body,
        out_type=jax.ShapeDtypeStruct((_NWORK, 16), jnp.float32),
        mesh=mesh,
        compiler_params=pltpu.CompilerParams(needs_layout_passes=False),
        scratch_types=[
            pltpu.VMEM((_N,), jnp.int32),
            pltpu.VMEM((_N,), jnp.int32),
            pltpu.VMEM((_N,), jnp.float32),
            pltpu.VMEM((_CH * _W // 1024, 8, 128), jnp.float32),
            pltpu.VMEM((16,), jnp.float32),
            pltpu.SemaphoreType.DMA,
            pltpu.SemaphoreType.DMA,
        ],
    )(pred2, tgt_t, station_rows, station_cols)
    return jnp.sum(partials) / (_N * _B)


# speedup vs baseline: 1.0733x; 1.0733x over previous
import jax
import jax.numpy as jnp
from jax import lax
from jax.experimental import pallas as pl
from jax.experimental.pallas import tpu as pltpu
from jax.experimental.pallas import tpu_sc as plsc

_B = 16
_H = 512
_W = 512
_N = 2000
_NWORK = 32
_ROWS_PER_W = (_B * _H) // _NWORK
_CH = 128
_NCHUNK = _ROWS_PER_W // _CH
_NVEC = _N // 16


def _station_loss_body(pred_hbm, tgt_hbm, rows_hbm, cols_hbm, out_hbm,
                       rows_v, cols_v, tgtb_v, slab_v, acc_v,
                       sem, slab_sem):
    cid = lax.axis_index("c")
    sid = lax.axis_index("s")
    wid = sid * 2 + cid
    b = wid // 2
    row0 = wid * _ROWS_PER_W
    h0 = (wid % 2) * _ROWS_PER_W

    cp_slab0 = pltpu.async_copy(
        pred_hbm.at[pl.ds(row0, _CH), :], slab_v, slab_sem)
    cp_r = pltpu.async_copy(rows_hbm, rows_v, sem)
    cp_c = pltpu.async_copy(cols_hbm, cols_v, sem)
    cp_t = pltpu.async_copy(tgt_hbm.at[pl.ds(b * _N, _N)], tgtb_v, sem)
    cp_r.wait()
    cp_c.wait()
    cp_t.wait()

    def pre_body(i, carry):
        base = i * 16
        r = rows_v[pl.ds(base, 16)]
        c = cols_v[pl.ds(base, 16)]
        rows_v[pl.ds(base, 16)] = ((r - (h0 - _ROWS_PER_W)) << 9) | c
        return carry

    lax.fori_loop(0, _NVEC, pre_body, 0, unroll=4)
    cp_slab0.wait()

    acc = jnp.zeros((16,), jnp.float32)
    for k in range(_NCHUNK):
        if k > 0:
            pltpu.sync_copy(pred_hbm.at[pl.ds(row0 + k * _CH, _CH), :],
                            slab_v)

        def sweep_body(i, acc, k=k):
            base = i * 16
            pk = rows_v[pl.ds(base, 16)]
            m = (pk >> 16) == (_ROWS_PER_W >> 7) + k
            x = plsc.load_gather(
                slab_v, [(pk >> 9) & (_CH - 1), pk & (_W - 1)], mask=m)
            d = jnp.abs(x - tgtb_v[pl.ds(base, 16)])
            return acc + jnp.where(m, d, 0.0)

        acc = lax.fori_loop(0, _NVEC, sweep_body, acc, unroll=8)

    acc_v[...] = acc
    pltpu.sync_copy(acc_v, out_hbm.at[wid])


def kernel(pred_images, target_runoff_values, station_rows, station_cols):
    pred2 = pred_images.reshape(_B * _H, _W)
    tgt_t = target_runoff_values[:, :_B].T.reshape(-1)

    mesh = plsc.VectorSubcoreMesh(core_axis_name="c", subcore_axis_name="s")
    partials = pl.kernel(
        _station_loss_body,
        out_type=jax.ShapeDtypeStruct((_NWORK, 16), jnp.float32),
        mesh=mesh,
        compiler_params=pltpu.CompilerParams(needs_layout_passes=False),
        scratch_types=[
            pltpu.VMEM((_N,), jnp.int32),
            pltpu.VMEM((_N,), jnp.int32),
            pltpu.VMEM((_N,), jnp.float32),
            pltpu.VMEM((_CH, _W), jnp.float32),
            pltpu.VMEM((16,), jnp.float32),
            pltpu.SemaphoreType.DMA,
            pltpu.SemaphoreType.DMA,
        ],
    )(pred2, tgt_t, station_rows, station_cols)
    return jnp.sum(partials) / (_N * _B)
